# baseline (speedup 1.0000x reference)
import numpy as np

import jax
import jax.numpy as jnp
from jax import lax
from jax.experimental import pallas as pl
from jax.experimental.pallas import tpu as pltpu
from jax.experimental.pallas import tpu_sc as plsc

ROWS = 128
N = 32768
L = 16
NV = N // L
NC = 2
NS = 16
NW = NC * NS
RPW = ROWS // NW

NB = 2048
SHIFTS = (0, 11, 22)
MASKS = (2047, 2047, 1023)
NBINS = (2048, 2048, 1024)

C = 4
CV = NV // C
CH_SHIFT = 13
UP = 4
U0 = 16

MININT = np.int32(-2147483648)


def _to_key(v):
    m = v >> 31
    return v ^ (m | MININT)


def _from_key(k):
    m = k >> 31
    return k ^ (~m | MININT)


def _digit(k, p):
    return lax.shift_right_logical(k, jnp.int32(SHIFTS[p])) & jnp.int32(MASKS[p])


def _zero(ref, n):
    zeros = jnp.zeros((L,), jnp.int32)

    def body(i, c):
        ref[pl.ds(i * L, L)] = zeros
        return c

    lax.fori_loop(0, n // L, body, 0)


def _body(x_hbm, out_hbm, buf_a, buf_b, cnt_a, cnt_b, gtmp,
          ptr_0, ptr_1, ptr_2, ptr_3):
    wid = lax.axis_index("s") * NC + lax.axis_index("c")
    ptrs = (ptr_0, ptr_1, ptr_2, ptr_3)
    zeros = jnp.zeros((L,), jnp.int32)

    _zero(cnt_a, C * NB)
    _zero(cnt_b, C * NB)

    def build_ptrs(cnt, nbins):
        def totals(i, c):
            ds = pl.ds(i * L, L)
            tot = cnt[pl.ds(i * L, L)]
            for cix in range(1, C):
                tot = tot + cnt[pl.ds(cix * NB + i * L, L)]
            gtmp[ds] = tot
            return c

        lax.fori_loop(0, nbins // L, totals, 0)

        def excl(i, carry):
            ds = pl.ds(i * L, L)
            h = gtmp[ds]
            inc = plsc.cumsum(h)
            gtmp[ds] = inc - h + carry
            return carry + jnp.sum(h)

        lax.fori_loop(0, nbins // L, excl, jnp.int32(0))

        def bases(i, c):
            ds = pl.ds(i * L, L)
            run = gtmp[ds]
            for cix in range(C):
                ptrs[cix][ds] = run
                run = run + cnt[pl.ds(cix * NB + i * L, L)]
                cnt[pl.ds(cix * NB + i * L, L)] = zeros
            return c

        lax.fori_loop(0, nbins // L, bases, 0)

    def row_body(r, c0):
        row = wid * RPW + r

        pltpu.sync_copy(x_hbm.at[row], buf_a)

        def sweep0(i, c):
            slots = [(cix, pl.ds((cix * CV + i * (U0 // C) + j) * L, L))
                     for j in range(U0 // C) for cix in range(C)]
            ks = []
            for _, sl in slots:
                k = _to_key(buf_a[sl])
                buf_a[sl] = k
                ks.append(k)
            digs = [_digit(k, 0) for k in ks]
            scans = [plsc.scan_count(d) for d in digs]
            for (cix, _), d, (cnt, lastm) in zip(slots, digs, scans):
                plsc.addupdate_scatter(cnt_a, [d + jnp.int32(cix * NB)], cnt,
                                       mask=lastm)
            return c

        lax.fori_loop(0, CV // (U0 // C), sweep0, 0)

        def permute(p, src, dst, cnt_cur, cnt_next):
            build_ptrs(cnt_cur, NBINS[p])

            def sweep(i, c):
                slots = [(cix, cix * CV + i * UP + j)
                         for j in range(UP) for cix in range(C)]
                ks = [src[pl.ds(iv * L, L)] for _, iv in slots]
                digs = [_digit(k, p) for k in ks]
                scans = [plsc.scan_count(d) for d in digs]
                vals = ks if p < 2 else [_from_key(k) for k in ks]
                offs = []
                for s, (cix, _) in enumerate(slots):
                    cnt, lastm = scans[s]
                    d = digs[s]
                    base = plsc.load_gather(ptrs[cix], [d])
                    off = base + cnt - 1
                    plsc.store_scatter(dst, [off], vals[s])
                    plsc.store_scatter(ptrs[cix], [d], base + cnt, mask=lastm)
                    offs.append(off)
                if cnt_next is not None:
                    idx2s = []
                    for s, k in enumerate(ks):
                        d2 = _digit(k, p + 1)
                        idx2 = lax.shift_left(
                            lax.shift_right_logical(offs[s],
                                                    jnp.int32(CH_SHIFT)),
                            jnp.int32(11)) | d2
                        idx2s.append(idx2)
                    scans2 = [plsc.scan_count(ix) for ix in idx2s]
                    for ix, (cnt2, last2) in zip(idx2s, scans2):
                        plsc.addupdate_scatter(cnt_next, [ix], cnt2,
                                               mask=last2)
                return c

            lax.fori_loop(0, CV // UP, sweep, 0)

        permute(0, buf_a, buf_b, cnt_a, cnt_b)
        permute(1, buf_b, buf_a, cnt_b, cnt_a)
        permute(2, buf_a, buf_b, cnt_a, None)

        pltpu.sync_copy(buf_b, out_hbm.at[row])
        return c0

    lax.fori_loop(0, RPW, row_body, 0)


@jax.jit
def kernel(x):
    xi = lax.bitcast_convert_type(x, jnp.int32)
    mesh = plsc.VectorSubcoreMesh(core_axis_name="c", subcore_axis_name="s")
    sort_rows = pl.kernel(
        _body,
        out_type=jax.ShapeDtypeStruct((ROWS, N), jnp.int32),
        mesh=mesh,
        compiler_params=pltpu.CompilerParams(needs_layout_passes=False),
        scratch_types=[
            pltpu.VMEM((N,), jnp.int32),
            pltpu.VMEM((N,), jnp.int32),
            pltpu.VMEM((C * NB,), jnp.int32),
            pltpu.VMEM((C * NB,), jnp.int32),
            pltpu.VMEM((NB,), jnp.int32),
        ] + [pltpu.VMEM((NB,), jnp.int32) for _ in range(C)],
    )
    oi = sort_rows(xi)
    return lax.bitcast_convert_type(oi, jnp.float32)


# speedup vs baseline: 1.1321x; 1.1321x over previous
import numpy as np

import jax
import jax.numpy as jnp
from jax import lax
from jax.experimental import pallas as pl
from jax.experimental.pallas import tpu as pltpu
from jax.experimental.pallas import tpu_sc as plsc

ROWS = 128
N = 32768
L = 16
NV = N // L
NC = 2
NS = 16
NW = NC * NS
RPW = ROWS // NW

NB = 2048
SHIFTS = (0, 11, 22)
MASKS = (2047, 2047, 1023)
NBINS = (2048, 2048, 1024)

MININT = np.int32(-2147483648)


def _to_key(v):
    m = v >> 31
    return v ^ (m | MININT)


def _from_key(k):
    m = k >> 31
    return k ^ (~m | MININT)


def _digit(k, p):
    return lax.shift_right_logical(k, jnp.int32(SHIFTS[p])) & jnp.int32(MASKS[p])


def _zero_hist(hist, nbins):
    zeros = jnp.zeros((L,), jnp.int32)

    def body(i, c):
        hist[pl.ds(i * L, L)] = zeros
        return c

    lax.fori_loop(0, nbins // L, body, 0)


def _exclusive_scan(hist, nbins):
    def body(i, carry):
        h = hist[pl.ds(i * L, L)]
        inc = plsc.cumsum(h)
        hist[pl.ds(i * L, L)] = inc - h + carry
        return carry + jnp.sum(h)

    lax.fori_loop(0, nbins // L, body, jnp.int32(0))


UNROLL = 16


def _body(x_hbm, out_hbm, buf_a, buf_b, buf_c, hist_0, hist_1, hist_2,
          sem_in, sem_out):
    wid = lax.axis_index("s") * NC + lax.axis_index("c")
    hists = (hist_0, hist_1, hist_2)
    bufs = (buf_a, buf_b, buf_c)
    row0 = wid * RPW

    def sort_row(src0, pong):
        for p in range(3):
            _zero_hist(hists[p], NBINS[p])

        def sweep0(i, c):
            ks = []
            for u in range(UNROLL):
                v = src0[pl.ds((i * UNROLL + u) * L, L)]
                ks.append(_to_key(v))
            digs = [[_digit(k, p) for k in ks] for p in range(3)]
            for p in range(3):
                scans = [plsc.scan_count(d) for d in digs[p]]
                for u in range(UNROLL):
                    cnt, lastm = scans[u]
                    plsc.addupdate_scatter(hists[p], [digs[p][u]], cnt,
                                           mask=lastm)
            return c

        lax.fori_loop(0, NV // UNROLL, sweep0, 0)

        def permute(p, src, dst):
            hist = hists[p]
            _exclusive_scan(hist, NBINS[p])

            def sweep(i, c):
                raw = [src[pl.ds((i * UNROLL + u) * L, L)]
                       for u in range(UNROLL)]
                ks = [_to_key(v) for v in raw] if p == 0 else raw
                digs = [_digit(k, p) for k in ks]
                scans = [plsc.scan_count(d) for d in digs]
                vals = ks if p < 2 else [_from_key(k) for k in ks]
                for u in range(UNROLL):
                    cnt, lastm = scans[u]
                    d = digs[u]
                    base = plsc.load_gather(hist, [d])
                    off = base + cnt - 1
                    plsc.store_scatter(dst, [off], vals[u])
                    plsc.store_scatter(hist, [d], base + cnt, mask=lastm)
                return c

            lax.fori_loop(0, NV // UNROLL, sweep, 0)

        permute(0, src0, pong)
        permute(1, pong, src0)
        permute(2, src0, pong)

    sched_x = [0, 2, 1, 0]
    sched_y = [1, 0, 2, 1]
    in_h = {0: pltpu.async_copy(x_hbm.at[row0], bufs[0], sem_in)}
    out_h = {}
    for r in range(RPW):
        x_buf = bufs[sched_x[r]]
        y_buf = bufs[sched_y[r]]
        in_h[r].wait()
        if r >= 1:
            out_h[r - 1].wait()
        if r + 1 < RPW:
            in_h[r + 1] = pltpu.async_copy(
                x_hbm.at[row0 + (r + 1)], bufs[sched_x[r + 1]], sem_in)
        sort_row(x_buf, y_buf)
        out_h[r] = pltpu.async_copy(y_buf, out_hbm.at[row0 + r], sem_out)
    out_h[RPW - 1].wait()


@jax.jit
def kernel(x):
    xi = lax.bitcast_convert_type(x, jnp.int32)
    mesh = plsc.VectorSubcoreMesh(core_axis_name="c", subcore_axis_name="s")
    sort_rows = pl.kernel(
        _body,
        out_type=jax.ShapeDtypeStruct((ROWS, N), jnp.int32),
        mesh=mesh,
        compiler_params=pltpu.CompilerParams(needs_layout_passes=False),
        scratch_types=[
            pltpu.VMEM((N,), jnp.int32),
            pltpu.VMEM((N,), jnp.int32),
            pltpu.VMEM((N,), jnp.int32),
            pltpu.VMEM((NBINS[0],), jnp.int32),
            pltpu.VMEM((NBINS[1],), jnp.int32),
            pltpu.VMEM((NBINS[2],), jnp.int32),
            pltpu.SemaphoreType.DMA,
            pltpu.SemaphoreType.DMA,
        ],
    )
    oi = sort_rows(xi)
    return lax.bitcast_convert_type(oi, jnp.float32)


# speedup vs baseline: 1.1588x; 1.0236x over previous
import numpy as np

import jax
import jax.numpy as jnp
from jax import lax
from jax.experimental import pallas as pl
from jax.experimental.pallas import tpu as pltpu
from jax.experimental.pallas import tpu_sc as plsc

ROWS = 128
N = 32768
L = 16
NV = N // L
NC = 2
NS = 16
NW = NC * NS
RPW = ROWS // NW

NB = 2048
SHIFTS = (0, 11, 22)
MASKS = (2047, 2047, 1023)
NBINS = (2048, 2048, 1024)

MININT = np.int32(-2147483648)


def _to_key(v):
    m = v >> 31
    return v ^ (m | MININT)


def _from_key(k):
    m = k >> 31
    return k ^ (~m | MININT)


def _digit(k, p):
    return lax.shift_right_logical(k, jnp.int32(SHIFTS[p])) & jnp.int32(MASKS[p])


def _zero_hist(hist, nbins):
    zeros = jnp.zeros((L,), jnp.int32)

    def body(i, c):
        hist[pl.ds(i * L, L)] = zeros
        return c

    lax.fori_loop(0, nbins // L, body, 0)


def _exclusive_scan(hist, nbins):
    def body(i, carry):
        h = hist[pl.ds(i * L, L)]
        inc = plsc.cumsum(h)
        hist[pl.ds(i * L, L)] = inc - h + carry
        return carry + jnp.sum(h)

    lax.fori_loop(0, nbins // L, body, jnp.int32(0))


UNROLL = 16


def _body(x_hbm, out_hbm, buf_a, buf_b, buf_c,
          h0a, h1a, h2a, h0b, h1b, h2b, sem_in, sem_out):
    wid = lax.axis_index("s") * NC + lax.axis_index("c")
    hist_sets = ((h0a, h1a, h2a), (h0b, h1b, h2b))
    bufs = (buf_a, buf_b, buf_c)
    row0 = wid * RPW

    def count_all(src0, hists):
        for p in range(3):
            _zero_hist(hists[p], NBINS[p])

        def sweep0(i, c):
            ks = []
            for u in range(UNROLL):
                v = src0[pl.ds((i * UNROLL + u) * L, L)]
                ks.append(_to_key(v))
            digs = [[_digit(k, p) for k in ks] for p in range(3)]
            for p in range(3):
                scans = [plsc.scan_count(d) for d in digs[p]]
                for u in range(UNROLL):
                    cnt, lastm = scans[u]
                    plsc.addupdate_scatter(hists[p], [digs[p][u]], cnt,
                                           mask=lastm)
            return c

        lax.fori_loop(0, NV // UNROLL, sweep0, 0)

    def permute(p, src, dst, hist, count_src, count_ps, count_hists):
        _exclusive_scan(hist, NBINS[p])

        def sweep(i, c):
            raw = [src[pl.ds((i * UNROLL + u) * L, L)]
                   for u in range(UNROLL)]
            ks = [_to_key(v) for v in raw] if p == 0 else raw
            digs = [_digit(k, p) for k in ks]
            scans = [plsc.scan_count(d) for d in digs]
            if count_src is not None:
                ks2 = [_to_key(count_src[pl.ds((i * UNROLL + u) * L, L)])
                       for u in range(UNROLL)]
                digs2 = [[_digit(k2, cp) for k2 in ks2] for cp in count_ps]
                scans2 = [[plsc.scan_count(d2) for d2 in dd] for dd in digs2]
            vals = ks if p < 2 else [_from_key(k) for k in ks]
            for u in range(UNROLL):
                cnt, lastm = scans[u]
                d = digs[u]
                base = plsc.load_gather(hist, [d])
                off = base + cnt - 1
                plsc.store_scatter(dst, [off], vals[u])
                plsc.store_scatter(hist, [d], base + cnt, mask=lastm)
            if count_src is not None:
                for ci, ch in enumerate(count_hists):
                    for u in range(UNROLL):
                        cnt2, lastm2 = scans2[ci][u]
                        plsc.addupdate_scatter(ch, [digs2[ci][u]], cnt2,
                                               mask=lastm2)
            return c

        lax.fori_loop(0, NV // UNROLL, sweep, 0)

    sched_x = [0, 2, 1, 0]
    sched_y = [1, 0, 2, 1]
    in_h = {0: pltpu.async_copy(x_hbm.at[row0], bufs[0], sem_in)}
    out_h = {}
    in_h[0].wait()
    count_all(bufs[0], hist_sets[0])
    for r in range(RPW):
        x_buf = bufs[sched_x[r]]
        y_buf = bufs[sched_y[r]]
        hists = hist_sets[r % 2]
        nhists = hist_sets[(r + 1) % 2]
        if r >= 1:
            out_h[r - 1].wait()
        if r + 1 < RPW:
            in_h[r + 1] = pltpu.async_copy(
                x_hbm.at[row0 + (r + 1)], bufs[sched_x[r + 1]], sem_in)
        last = r + 1 >= RPW
        permute(0, x_buf, y_buf, hists[0], None, (), ())
        if not last:
            in_h[r + 1].wait()
            for p in range(3):
                _zero_hist(nhists[p], NBINS[p])
            nxt = bufs[sched_x[r + 1]]
            permute(1, y_buf, x_buf, hists[1], nxt, (0,), (nhists[0],))
            permute(2, x_buf, y_buf, hists[2], nxt, (1, 2),
                    (nhists[1], nhists[2]))
        else:
            permute(1, y_buf, x_buf, hists[1], None, (), ())
            permute(2, x_buf, y_buf, hists[2], None, (), ())
        out_h[r] = pltpu.async_copy(y_buf, out_hbm.at[row0 + r], sem_out)
    out_h[RPW - 1].wait()


@jax.jit
def kernel(x):
    xi = lax.bitcast_convert_type(x, jnp.int32)
    mesh = plsc.VectorSubcoreMesh(core_axis_name="c", subcore_axis_name="s")
    sort_rows = pl.kernel(
        _body,
        out_type=jax.ShapeDtypeStruct((ROWS, N), jnp.int32),
        mesh=mesh,
        compiler_params=pltpu.CompilerParams(needs_layout_passes=False),
        scratch_types=[
            pltpu.VMEM((N,), jnp.int32),
            pltpu.VMEM((N,), jnp.int32),
            pltpu.VMEM((N,), jnp.int32),
            pltpu.VMEM((NBINS[0],), jnp.int32),
            pltpu.VMEM((NBINS[1],), jnp.int32),
            pltpu.VMEM((NBINS[2],), jnp.int32),
            pltpu.VMEM((NBINS[0],), jnp.int32),
            pltpu.VMEM((NBINS[1],), jnp.int32),
            pltpu.VMEM((NBINS[2],), jnp.int32),
            pltpu.SemaphoreType.DMA,
            pltpu.SemaphoreType.DMA,
        ],
    )
    oi = sort_rows(xi)
    return lax.bitcast_convert_type(oi, jnp.float32)
